# Initial kernel scaffold; baseline (speedup 1.0000x reference)
import jax
import jax.numpy as jnp
from jax.experimental import pallas as pl


def kernel(x, positions, enc_W0, enc_b0, enc_W1, enc_b1, msg_W, msg_b, upd_W, upd_b, proj_W0, proj_b0, proj_W1, proj_b1):
    raise NotImplementedError("write your pallas kernel here")


# speedup vs baseline: 5.0608x; 5.0608x over previous
import functools

import jax
import jax.numpy as jnp
from jax import lax
from jax.experimental import pallas as pl
from jax.experimental.pallas import tpu as pltpu
from jax.experimental.pallas import tpu_sc as plsc

B = 2
N = 4096
NODE_DIM = 16
HIDDEN = 128
PROJ = 256
K = 16
N_MP = 3
NT = B * N

F32 = jnp.float32
I32 = jnp.int32

RB = 256

NC, NS = 2, 16
NW = NC * NS
NPW = NT // NW
EPW = NPW * K
ECH = 64
NCHK = EPW // ECH

_dot = functools.partial(jnp.dot, preferred_element_type=F32,
                         precision=lax.Precision.HIGHEST)


def _topk_body(posr_ref, posc_ref, out_ref):
    b = pl.program_id(0)
    rb = pl.program_id(1)
    d2 = jnp.zeros((RB, N), F32)
    for d in range(3):
        pr = posr_ref[0, d, :].reshape(RB, 1)
        pc = posc_ref[0, d, :].reshape(1, N)
        delta = jnp.abs(pr - pc)
        delta = jnp.where(delta > 0.5, 1.0 - delta, delta)
        d2 = d2 + delta * delta
    col = lax.broadcasted_iota(I32, (RB, N), 1)
    rowg = rb * RB + lax.broadcasted_iota(I32, (RB, N), 0)
    inf = jnp.float32(jnp.inf)
    d2 = jnp.where(col == rowg, inf, d2)
    kio = lax.broadcasted_iota(I32, (RB, K), 1)
    picks0 = jnp.zeros((RB, K), I32)

    def step(t, carry):
        d2, picks = carry
        m = jnp.min(d2, axis=1, keepdims=True)
        cand = jnp.where(d2 == m, col, jnp.int32(2**30))
        pick = jnp.min(cand, axis=1, keepdims=True)
        picks = jnp.where(kio == t, pick, picks)
        d2 = jnp.where(col == pick, inf, d2)
        return d2, picks

    _, picks = lax.fori_loop(0, K, step, (d2, picks0))
    out_ref[0] = picks + b * N


def _topk(pos_t):
    return pl.pallas_call(
        _topk_body,
        grid=(B, N // RB),
        in_specs=[
            pl.BlockSpec((1, 3, RB), lambda b, r: (b, 0, r)),
            pl.BlockSpec((1, 3, N), lambda b, r: (b, 0, 0)),
        ],
        out_specs=pl.BlockSpec((1, RB, K), lambda b, r: (b, r, 0)),
        out_shape=jax.ShapeDtypeStruct((B, N, K), I32),
    )(pos_t, pos_t)


def _enc_body(x_ref, w0_ref, b0_ref, w1_ref, b1_ref, ws_ref, bm_ref, wr_ref,
              h_ref, a_ref, c_ref):
    h = jnp.maximum(_dot(x_ref[...], w0_ref[...]) + b0_ref[...], 0.0)
    h = jnp.maximum(_dot(h, w1_ref[...]) + b1_ref[...], 0.0)
    h_ref[...] = h
    a_ref[...] = _dot(h, ws_ref[...]) + bm_ref[...]
    c_ref[...] = _dot(h, wr_ref[...])


def _encode(x, w0, b0, w1, b1, ws, bm, wr):
    full = lambda s: pl.BlockSpec(s, lambda r: tuple(0 for _ in s))
    row = lambda w: pl.BlockSpec((RB, w), lambda r: (r, 0))
    return pl.pallas_call(
        _enc_body,
        grid=(NT // RB,),
        in_specs=[row(NODE_DIM), full((NODE_DIM, HIDDEN)), full((1, HIDDEN)),
                  full((HIDDEN, HIDDEN)), full((1, HIDDEN)),
                  full((HIDDEN, HIDDEN)), full((1, HIDDEN)),
                  full((HIDDEN, HIDDEN))],
        out_specs=[row(HIDDEN), row(HIDDEN), row(HIDDEN)],
        out_shape=[jax.ShapeDtypeStruct((NT, HIDDEN), F32)] * 3,
    )(x, w0, b0.reshape(1, -1), w1, b1.reshape(1, -1), ws, bm.reshape(1, -1), wr)


def _sc_deg(dst_flat):
    mesh = plsc.VectorSubcoreMesh(core_axis_name="c", subcore_axis_name="s",
                                  num_cores=NC, num_subcores=NS)
    rows_per_sub = NT // NS

    @functools.partial(
        pl.kernel,
        out_type=jax.ShapeDtypeStruct((NC, NT, HIDDEN), F32),
        mesh=mesh,
        scratch_types=[
            pltpu.VMEM((ECH,), I32),
            pltpu.VMEM((ECH, HIDDEN), F32),
            pltpu.VMEM_SHARED((NT, HIDDEN), F32),
        ],
    )
    def k(dst_hbm, out_hbm, idx_v, ones_v, acc_sh):
        cid = lax.axis_index("c")
        sid = lax.axis_index("s")
        wid = cid * NS + sid
        zero16 = jnp.zeros((16,), F32)
        ones16 = jnp.ones((16,), F32)

        def zrow(i, _):
            for j in range(HIDDEN // 16):
                ones_v[i, pl.ds(16 * j, 16)] = zero16
            return 0
        lax.fori_loop(0, ECH, zrow, 0)
        nz = rows_per_sub // ECH
        def zacc(j, _):
            pltpu.sync_copy(ones_v, acc_sh.at[pl.ds(sid * rows_per_sub + j * ECH, ECH)])
            return 0
        lax.fori_loop(0, nz, zacc, 0)

        def orow(i, _):
            ones_v[i, pl.ds(0, 16)] = ones16
            return 0
        lax.fori_loop(0, ECH, orow, 0)
        plsc.subcore_barrier()

        def chunk(ch, _):
            ebase = wid * EPW + ch * ECH
            pltpu.sync_copy(dst_hbm.at[pl.ds(ebase, ECH)], idx_v)
            pltpu.sync_copy(ones_v, acc_sh.at[idx_v], add=True)
            return 0
        lax.fori_loop(0, NCHK, chunk, 0)
        plsc.subcore_barrier()

        base = sid * rows_per_sub
        pltpu.sync_copy(acc_sh.at[pl.ds(base, rows_per_sub)],
                        out_hbm.at[cid, pl.ds(base, rows_per_sub)])

    return k(dst_flat)


def _sc_agg(a, c, dst_flat):
    mesh = plsc.VectorSubcoreMesh(core_axis_name="c", subcore_axis_name="s",
                                  num_cores=NC, num_subcores=NS)
    nfc = HIDDEN // 16
    rows_per_sub = NT // NS

    @functools.partial(
        pl.kernel,
        out_type=jax.ShapeDtypeStruct((NC, NT, HIDDEN), F32),
        mesh=mesh,
        scratch_types=[
            pltpu.VMEM((NPW, HIDDEN), F32),
            pltpu.VMEM((ECH,), I32),
            pltpu.VMEM((ECH, HIDDEN), F32),
            pltpu.VMEM((ECH, HIDDEN), F32),
            pltpu.VMEM_SHARED((NT, HIDDEN), F32),
            pltpu.SemaphoreType.DMA,
        ],
    )
    def k(a_hbm, c_hbm, dst_hbm, out_hbm, a_v, idx_v, rows_v, msg_v, acc_sh, sem):
        cid = lax.axis_index("c")
        sid = lax.axis_index("s")
        wid = cid * NS + sid

        zero16 = jnp.zeros((16,), F32)

        def zrow(i, _):
            for j in range(nfc):
                msg_v[i, pl.ds(16 * j, 16)] = zero16
            return 0
        lax.fori_loop(0, ECH, zrow, 0)
        nz = rows_per_sub // ECH
        def zacc(j, _):
            pltpu.sync_copy(msg_v, acc_sh.at[pl.ds(sid * rows_per_sub + j * ECH, ECH)])
            return 0
        lax.fori_loop(0, nz, zacc, 0)
        plsc.subcore_barrier()

        pltpu.sync_copy(a_hbm.at[pl.ds(wid * NPW, NPW)], a_v)

        def chunk(ch, _):
            ebase = wid * EPW + ch * ECH
            pltpu.sync_copy(dst_hbm.at[pl.ds(ebase, ECH)], idx_v)
            pltpu.async_copy(c_hbm.at[idx_v], rows_v, sem).wait()

            def edge(e, _):
                n = (ch * ECH + e) // K
                for j in range(nfc):
                    av = a_v[n, pl.ds(16 * j, 16)]
                    cv = rows_v[e, pl.ds(16 * j, 16)]
                    msg_v[e, pl.ds(16 * j, 16)] = jnp.maximum(av + cv, 0.0)
                return 0
            lax.fori_loop(0, ECH, edge, 0)
            pltpu.sync_copy(msg_v, acc_sh.at[idx_v], add=True)
            return 0
        lax.fori_loop(0, NCHK, chunk, 0)
        plsc.subcore_barrier()

        base = sid * rows_per_sub
        pltpu.sync_copy(acc_sh.at[pl.ds(base, rows_per_sub)],
                        out_hbm.at[cid, pl.ds(base, rows_per_sub)])

    return k(a, c, dst_flat)


def _upd_agg(p0, p1, dinv):
    return (p0 + p1) * dinv


def _upd0_body(h_ref, p0_ref, p1_ref, d0_ref, d1_ref, uh_ref, ua_ref, bu_ref,
               ws_ref, bm_ref, wr_ref, h_out, a_out, c_out, dinv_out):
    cnt = d0_ref[:, 0:1] + d1_ref[:, 0:1]
    dinv = 1.0 / jnp.maximum(cnt, 1.0)
    agg = _upd_agg(p0_ref[...], p1_ref[...], dinv)
    h = h_ref[...]
    u = jnp.maximum(_dot(h, uh_ref[...]) + _dot(agg, ua_ref[...]) + bu_ref[...], 0.0)
    h = h + u
    h_out[...] = h
    a_out[...] = _dot(h, ws_ref[...]) + bm_ref[...]
    c_out[...] = _dot(h, wr_ref[...])
    dinv_out[...] = dinv[:, 0]


def _upd1_body(h_ref, p0_ref, p1_ref, dinv_ref, uh_ref, ua_ref, bu_ref,
               ws_ref, bm_ref, wr_ref, h_out, a_out, c_out):
    dinv = dinv_ref[...].reshape(RB, 1)
    agg = _upd_agg(p0_ref[...], p1_ref[...], dinv)
    h = h_ref[...]
    u = jnp.maximum(_dot(h, uh_ref[...]) + _dot(agg, ua_ref[...]) + bu_ref[...], 0.0)
    h = h + u
    h_out[...] = h
    a_out[...] = _dot(h, ws_ref[...]) + bm_ref[...]
    c_out[...] = _dot(h, wr_ref[...])


def _upd2_body(h_ref, p0_ref, p1_ref, dinv_ref, uh_ref, ua_ref, bu_ref,
               pw0_ref, pb0_ref, pw1_ref, pb1_ref, z_out, h_out):
    dinv = dinv_ref[...].reshape(RB, 1)
    agg = _upd_agg(p0_ref[...], p1_ref[...], dinv)
    h = h_ref[...]
    u = jnp.maximum(_dot(h, uh_ref[...]) + _dot(agg, ua_ref[...]) + bu_ref[...], 0.0)
    h = h + u
    h_out[...] = h
    t = jnp.maximum(_dot(h, pw0_ref[...]) + pb0_ref[...], 0.0)
    z_out[...] = _dot(t, pw1_ref[...]) + pb1_ref[...]


def _full(s):
    return pl.BlockSpec(s, lambda r: tuple(0 for _ in s))


def _row(w):
    return pl.BlockSpec((RB, w), lambda r: (r, 0))


_ROW1 = pl.BlockSpec((RB,), lambda r: (r,))


def _update0(h, p0, p1, d0, d1, uh, ua, bu, ws, bm, wr):
    return pl.pallas_call(
        _upd0_body,
        grid=(NT // RB,),
        in_specs=[_row(HIDDEN), _row(HIDDEN), _row(HIDDEN),
                  _row(HIDDEN), _row(HIDDEN),
                  _full((HIDDEN, HIDDEN)), _full((HIDDEN, HIDDEN)), _full((1, HIDDEN)),
                  _full((HIDDEN, HIDDEN)), _full((1, HIDDEN)), _full((HIDDEN, HIDDEN))],
        out_specs=[_row(HIDDEN), _row(HIDDEN), _row(HIDDEN), _ROW1],
        out_shape=[jax.ShapeDtypeStruct((NT, HIDDEN), F32)] * 3
        + [jax.ShapeDtypeStruct((NT,), F32)],
    )(h, p0, p1, d0, d1, uh, ua, bu.reshape(1, -1), ws, bm.reshape(1, -1), wr)


def _update1(h, p0, p1, dinv, uh, ua, bu, ws, bm, wr):
    return pl.pallas_call(
        _upd1_body,
        grid=(NT // RB,),
        in_specs=[_row(HIDDEN), _row(HIDDEN), _row(HIDDEN), _ROW1,
                  _full((HIDDEN, HIDDEN)), _full((HIDDEN, HIDDEN)), _full((1, HIDDEN)),
                  _full((HIDDEN, HIDDEN)), _full((1, HIDDEN)), _full((HIDDEN, HIDDEN))],
        out_specs=[_row(HIDDEN), _row(HIDDEN), _row(HIDDEN)],
        out_shape=[jax.ShapeDtypeStruct((NT, HIDDEN), F32)] * 3,
    )(h, p0, p1, dinv, uh, ua, bu.reshape(1, -1), ws, bm.reshape(1, -1), wr)


def _update2(h, p0, p1, dinv, uh, ua, bu, pw0, pb0, pw1, pb1):
    return pl.pallas_call(
        _upd2_body,
        grid=(NT // RB,),
        in_specs=[_row(HIDDEN), _row(HIDDEN), _row(HIDDEN), _ROW1,
                  _full((HIDDEN, HIDDEN)), _full((HIDDEN, HIDDEN)), _full((1, HIDDEN)),
                  _full((HIDDEN, HIDDEN)), _full((1, HIDDEN)),
                  _full((HIDDEN, PROJ)), _full((1, PROJ))],
        out_specs=[_row(PROJ), _row(HIDDEN)],
        out_shape=[jax.ShapeDtypeStruct((NT, PROJ), F32),
                   jax.ShapeDtypeStruct((NT, HIDDEN), F32)],
    )(h, p0, p1, dinv, uh, ua, bu.reshape(1, -1), pw0, pb0.reshape(1, -1),
      pw1, pb1.reshape(1, -1))


def kernel(x, positions, enc_W0, enc_b0, enc_W1, enc_b1, msg_W, msg_b,
           upd_W, upd_b, proj_W0, proj_b0, proj_W1, proj_b1):
    pos_t = positions.transpose(0, 2, 1)
    dst = _topk(pos_t)
    dst_flat = dst.reshape(NT * K)

    xf = x.reshape(NT, NODE_DIM)
    ws = [msg_W[i][:HIDDEN] for i in range(N_MP)]
    wr = [msg_W[i][HIDDEN:] for i in range(N_MP)]
    uh = [upd_W[i][:HIDDEN] for i in range(N_MP)]
    ua = [upd_W[i][HIDDEN:] for i in range(N_MP)]

    h, a, c = _encode(xf, enc_W0, enc_b0, enc_W1, enc_b1, ws[0], msg_b[0], wr[0])

    degp = _sc_deg(dst_flat)
    parts = _sc_agg(a, c, dst_flat)
    h, a, c, dinv = _update0(h, parts[0], parts[1], degp[0], degp[1],
                             uh[0], ua[0], upd_b[0], ws[1], msg_b[1], wr[1])

    parts = _sc_agg(a, c, dst_flat)
    h, a, c = _update1(h, parts[0], parts[1], dinv, uh[1], ua[1], upd_b[1],
                       ws[2], msg_b[2], wr[2])

    parts = _sc_agg(a, c, dst_flat)
    z, h = _update2(h, parts[0], parts[1], dinv, uh[2], ua[2], upd_b[2],
                    proj_W0, proj_b0, proj_W1, proj_b1)

    return z.reshape(B, N, PROJ), h.reshape(B, N, HIDDEN)


# speedup vs baseline: 5.9337x; 1.1725x over previous
import functools

import jax
import jax.numpy as jnp
from jax import lax
from jax.experimental import pallas as pl
from jax.experimental.pallas import tpu as pltpu
from jax.experimental.pallas import tpu_sc as plsc

B = 2
N = 4096
NODE_DIM = 16
HIDDEN = 128
PROJ = 256
K = 16
N_MP = 3
NT = B * N

F32 = jnp.float32
I32 = jnp.int32

RB = 256

NC, NS = 2, 16
NW = NC * NS
NPW = NT // NW
EPW = NPW * K
ECH = 64
NCHK = EPW // ECH
AECH = 64
ANPC = AECH // K
ANCHK = EPW // AECH

_dot = functools.partial(jnp.dot, preferred_element_type=F32,
                         precision=lax.Precision.HIGHEST)


def _topk_body(posr_ref, posc_ref, out_ref):
    b = pl.program_id(0)
    rb = pl.program_id(1)
    d2 = jnp.zeros((RB, N), F32)
    for d in range(3):
        pr = posr_ref[0, d, :].reshape(RB, 1)
        pc = posc_ref[0, d, :].reshape(1, N)
        delta = jnp.abs(pr - pc)
        delta = jnp.where(delta > 0.5, 1.0 - delta, delta)
        d2 = d2 + delta * delta
    col = lax.broadcasted_iota(I32, (RB, N), 1)
    rowg = rb * RB + lax.broadcasted_iota(I32, (RB, N), 0)
    inf = jnp.float32(jnp.inf)
    d2 = jnp.where(col == rowg, inf, d2)
    kio = lax.broadcasted_iota(I32, (RB, K), 1)
    picks0 = jnp.zeros((RB, K), I32)

    def step(t, carry):
        d2, picks = carry
        m = jnp.min(d2, axis=1, keepdims=True)
        cand = jnp.where(d2 == m, col, jnp.int32(2**30))
        pick = jnp.min(cand, axis=1, keepdims=True)
        picks = jnp.where(kio == t, pick, picks)
        d2 = jnp.where(col == pick, inf, d2)
        return d2, picks

    _, picks = lax.fori_loop(0, K, step, (d2, picks0))
    out_ref[0] = picks + b * N


def _topk(pos_t):
    return pl.pallas_call(
        _topk_body,
        grid=(B, N // RB),
        in_specs=[
            pl.BlockSpec((1, 3, RB), lambda b, r: (b, 0, r)),
            pl.BlockSpec((1, 3, N), lambda b, r: (b, 0, 0)),
        ],
        out_specs=pl.BlockSpec((1, RB, K), lambda b, r: (b, r, 0)),
        out_shape=jax.ShapeDtypeStruct((B, N, K), I32),
    )(pos_t, pos_t)


def _enc_body(x_ref, w0_ref, b0_ref, w1_ref, b1_ref, ws_ref, bm_ref, wr_ref,
              h_ref, a_ref, c_ref):
    h = jnp.maximum(_dot(x_ref[...], w0_ref[...]) + b0_ref[...], 0.0)
    h = jnp.maximum(_dot(h, w1_ref[...]) + b1_ref[...], 0.0)
    h_ref[...] = h
    a_ref[...] = _dot(h, ws_ref[...]) + bm_ref[...]
    c_ref[...] = _dot(h, wr_ref[...])


def _encode(x, w0, b0, w1, b1, ws, bm, wr):
    full = lambda s: pl.BlockSpec(s, lambda r: tuple(0 for _ in s))
    row = lambda w: pl.BlockSpec((RB, w), lambda r: (r, 0))
    return pl.pallas_call(
        _enc_body,
        grid=(NT // RB,),
        in_specs=[row(NODE_DIM), full((NODE_DIM, HIDDEN)), full((1, HIDDEN)),
                  full((HIDDEN, HIDDEN)), full((1, HIDDEN)),
                  full((HIDDEN, HIDDEN)), full((1, HIDDEN)),
                  full((HIDDEN, HIDDEN))],
        out_specs=[row(HIDDEN), row(HIDDEN), row(HIDDEN)],
        out_shape=[jax.ShapeDtypeStruct((NT, HIDDEN), F32)] * 3,
    )(x, w0, b0.reshape(1, -1), w1, b1.reshape(1, -1), ws, bm.reshape(1, -1), wr)


def _sc_deg(dst_flat):
    mesh = plsc.VectorSubcoreMesh(core_axis_name="c", subcore_axis_name="s",
                                  num_cores=NC, num_subcores=NS)
    rows_per_sub = NT // NS

    @functools.partial(
        pl.kernel,
        out_type=jax.ShapeDtypeStruct((NC, NT, HIDDEN), F32),
        mesh=mesh,
        scratch_types=[
            pltpu.VMEM((ECH,), I32),
            pltpu.VMEM((ECH, HIDDEN), F32),
            pltpu.VMEM_SHARED((NT, HIDDEN), F32),
        ],
    )
    def k(dst_hbm, out_hbm, idx_v, ones_v, acc_sh):
        cid = lax.axis_index("c")
        sid = lax.axis_index("s")
        wid = cid * NS + sid
        zero16 = jnp.zeros((16,), F32)
        ones16 = jnp.ones((16,), F32)

        def zrow(i, _):
            for j in range(HIDDEN // 16):
                ones_v[i, pl.ds(16 * j, 16)] = zero16
            return 0
        lax.fori_loop(0, ECH, zrow, 0)
        nz = rows_per_sub // ECH
        def zacc(j, _):
            pltpu.sync_copy(ones_v, acc_sh.at[pl.ds(sid * rows_per_sub + j * ECH, ECH)])
            return 0
        lax.fori_loop(0, nz, zacc, 0)

        def orow(i, _):
            ones_v[i, pl.ds(0, 16)] = ones16
            return 0
        lax.fori_loop(0, ECH, orow, 0)
        plsc.subcore_barrier()

        def chunk(ch, _):
            ebase = wid * EPW + ch * ECH
            pltpu.sync_copy(dst_hbm.at[pl.ds(ebase, ECH)], idx_v)
            pltpu.sync_copy(ones_v, acc_sh.at[idx_v], add=True)
            return 0
        lax.fori_loop(0, NCHK, chunk, 0)
        plsc.subcore_barrier()

        base = sid * rows_per_sub
        pltpu.sync_copy(acc_sh.at[pl.ds(base, rows_per_sub)],
                        out_hbm.at[cid, pl.ds(base, rows_per_sub)])

    return k(dst_flat)


def _sc_agg(a, c, dst3):
    mesh = plsc.VectorSubcoreMesh(core_axis_name="c", subcore_axis_name="s",
                                  num_cores=NC, num_subcores=NS)
    nfc = HIDDEN // 16
    rows_per_sub = NT // NS

    @functools.partial(
        pl.kernel,
        out_type=jax.ShapeDtypeStruct((NC, NT, HIDDEN), F32),
        mesh=mesh,
        scratch_types=[
            pltpu.VMEM((ANPC, HIDDEN), F32),
            pltpu.VMEM((ANPC, HIDDEN), F32),
            pltpu.VMEM((ANCHK, AECH), I32),
            pltpu.VMEM((AECH, HIDDEN), F32),
            pltpu.VMEM((AECH, HIDDEN), F32),
            pltpu.VMEM((AECH, HIDDEN), F32),
            pltpu.VMEM((AECH, HIDDEN), F32),
            pltpu.VMEM_SHARED((NT, HIDDEN), F32),
            pltpu.SemaphoreType.DMA,
            pltpu.SemaphoreType.DMA,
            pltpu.SemaphoreType.DMA,
            pltpu.SemaphoreType.DMA,
            pltpu.SemaphoreType.DMA,
            pltpu.SemaphoreType.DMA,
        ],
    )
    def k(a_hbm, c_hbm, dst_hbm, out_hbm, av0, av1, idx_all, rows0, rows1,
          msg0, msg1, acc_sh, gs0, gs1, ss0, ss1, as0, as1):
        cid = lax.axis_index("c")
        sid = lax.axis_index("s")
        wid = cid * NS + sid
        avs = (av0, av1)
        rows = (rows0, rows1)
        msgs = (msg0, msg1)
        gsems = (gs0, gs1)
        ssems = (ss0, ss1)
        asems = (as0, as1)
        nbase = wid * NPW

        zero16 = jnp.zeros((16,), F32)

        def zrow(i, _):
            for j in range(nfc):
                msg0[i, pl.ds(16 * j, 16)] = zero16
            return 0
        lax.fori_loop(0, AECH, zrow, 0)
        def zacc(j, _):
            pltpu.sync_copy(msg0, acc_sh.at[pl.ds(sid * rows_per_sub + j * AECH, AECH)])
            return 0
        lax.fori_loop(0, rows_per_sub // AECH, zacc, 0)
        plsc.subcore_barrier()

        pltpu.sync_copy(dst_hbm.at[wid], idx_all)

        pltpu.async_copy(c_hbm.at[idx_all.at[0]], rows0, gs0)
        pltpu.async_copy(c_hbm.at[idx_all.at[1]], rows1, gs1)
        pltpu.async_copy(a_hbm.at[pl.ds(nbase, ANPC)], av0, as0)
        pltpu.async_copy(a_hbm.at[pl.ds(nbase + ANPC, ANPC)], av1, as1)

        def pair(g2, _):
            for b in range(2):
                ch = g2 * 2 + b
                rv, mv, av = rows[b], msgs[b], avs[b]
                gsem, ssem, asem = gsems[b], ssems[b], asems[b]
                pltpu.make_async_copy(c_hbm.at[idx_all.at[ch]], rv, gsem).wait()
                pltpu.make_async_copy(a_hbm.at[pl.ds(0, ANPC)], av, asem).wait()

                @pl.when(g2 > 0)
                def _wait_prev_scatter():
                    pltpu.make_async_copy(mv, acc_sh.at[idx_all.at[ch]], ssem).wait()

                def edge(e, _):
                    n = e // K
                    for j in range(nfc):
                        cv = rv[e, pl.ds(16 * j, 16)]
                        mv[e, pl.ds(16 * j, 16)] = jnp.maximum(av[n, pl.ds(16 * j, 16)] + cv, 0.0)
                    return 0
                lax.fori_loop(0, AECH, edge, 0)

                pltpu.async_copy(mv, acc_sh.at[idx_all.at[ch]], ssem, add=True)

                @pl.when(ch + 2 < ANCHK)
                def _next_prefetch():
                    pltpu.async_copy(c_hbm.at[idx_all.at[ch + 2]], rv, gsem)
                    pltpu.async_copy(
                        a_hbm.at[pl.ds(nbase + (ch + 2) * ANPC, ANPC)], av, asem)
            return 0
        lax.fori_loop(0, ANCHK // 2, pair, 0)

        pltpu.make_async_copy(msg0, acc_sh.at[idx_all.at[0]], ss0).wait()
        pltpu.make_async_copy(msg1, acc_sh.at[idx_all.at[1]], ss1).wait()
        plsc.subcore_barrier()

        base = sid * rows_per_sub
        pltpu.sync_copy(acc_sh.at[pl.ds(base, rows_per_sub)],
                        out_hbm.at[cid, pl.ds(base, rows_per_sub)])

    return k(a, c, dst3)


def _upd_agg(p0, p1, dinv):
    return (p0 + p1) * dinv


def _upd0_body(h_ref, p0_ref, p1_ref, d0_ref, d1_ref, uh_ref, ua_ref, bu_ref,
               ws_ref, bm_ref, wr_ref, h_out, a_out, c_out, dinv_out):
    cnt = d0_ref[:, 0:1] + d1_ref[:, 0:1]
    dinv = 1.0 / jnp.maximum(cnt, 1.0)
    agg = _upd_agg(p0_ref[...], p1_ref[...], dinv)
    h = h_ref[...]
    u = jnp.maximum(_dot(h, uh_ref[...]) + _dot(agg, ua_ref[...]) + bu_ref[...], 0.0)
    h = h + u
    h_out[...] = h
    a_out[...] = _dot(h, ws_ref[...]) + bm_ref[...]
    c_out[...] = _dot(h, wr_ref[...])
    dinv_out[...] = dinv[:, 0]


def _upd1_body(h_ref, p0_ref, p1_ref, dinv_ref, uh_ref, ua_ref, bu_ref,
               ws_ref, bm_ref, wr_ref, h_out, a_out, c_out):
    dinv = dinv_ref[...].reshape(RB, 1)
    agg = _upd_agg(p0_ref[...], p1_ref[...], dinv)
    h = h_ref[...]
    u = jnp.maximum(_dot(h, uh_ref[...]) + _dot(agg, ua_ref[...]) + bu_ref[...], 0.0)
    h = h + u
    h_out[...] = h
    a_out[...] = _dot(h, ws_ref[...]) + bm_ref[...]
    c_out[...] = _dot(h, wr_ref[...])


def _upd2_body(h_ref, p0_ref, p1_ref, dinv_ref, uh_ref, ua_ref, bu_ref,
               pw0_ref, pb0_ref, pw1_ref, pb1_ref, z_out, h_out):
    dinv = dinv_ref[...].reshape(RB, 1)
    agg = _upd_agg(p0_ref[...], p1_ref[...], dinv)
    h = h_ref[...]
    u = jnp.maximum(_dot(h, uh_ref[...]) + _dot(agg, ua_ref[...]) + bu_ref[...], 0.0)
    h = h + u
    h_out[...] = h
    t = jnp.maximum(_dot(h, pw0_ref[...]) + pb0_ref[...], 0.0)
    z_out[...] = _dot(t, pw1_ref[...]) + pb1_ref[...]


def _full(s):
    return pl.BlockSpec(s, lambda r: tuple(0 for _ in s))


def _row(w):
    return pl.BlockSpec((RB, w), lambda r: (r, 0))


_ROW1 = pl.BlockSpec((RB,), lambda r: (r,))


def _update0(h, p0, p1, d0, d1, uh, ua, bu, ws, bm, wr):
    return pl.pallas_call(
        _upd0_body,
        grid=(NT // RB,),
        in_specs=[_row(HIDDEN), _row(HIDDEN), _row(HIDDEN),
                  _row(HIDDEN), _row(HIDDEN),
                  _full((HIDDEN, HIDDEN)), _full((HIDDEN, HIDDEN)), _full((1, HIDDEN)),
                  _full((HIDDEN, HIDDEN)), _full((1, HIDDEN)), _full((HIDDEN, HIDDEN))],
        out_specs=[_row(HIDDEN), _row(HIDDEN), _row(HIDDEN), _ROW1],
        out_shape=[jax.ShapeDtypeStruct((NT, HIDDEN), F32)] * 3
        + [jax.ShapeDtypeStruct((NT,), F32)],
    )(h, p0, p1, d0, d1, uh, ua, bu.reshape(1, -1), ws, bm.reshape(1, -1), wr)


def _update1(h, p0, p1, dinv, uh, ua, bu, ws, bm, wr):
    return pl.pallas_call(
        _upd1_body,
        grid=(NT // RB,),
        in_specs=[_row(HIDDEN), _row(HIDDEN), _row(HIDDEN), _ROW1,
                  _full((HIDDEN, HIDDEN)), _full((HIDDEN, HIDDEN)), _full((1, HIDDEN)),
                  _full((HIDDEN, HIDDEN)), _full((1, HIDDEN)), _full((HIDDEN, HIDDEN))],
        out_specs=[_row(HIDDEN), _row(HIDDEN), _row(HIDDEN)],
        out_shape=[jax.ShapeDtypeStruct((NT, HIDDEN), F32)] * 3,
    )(h, p0, p1, dinv, uh, ua, bu.reshape(1, -1), ws, bm.reshape(1, -1), wr)


def _update2(h, p0, p1, dinv, uh, ua, bu, pw0, pb0, pw1, pb1):
    return pl.pallas_call(
        _upd2_body,
        grid=(NT // RB,),
        in_specs=[_row(HIDDEN), _row(HIDDEN), _row(HIDDEN), _ROW1,
                  _full((HIDDEN, HIDDEN)), _full((HIDDEN, HIDDEN)), _full((1, HIDDEN)),
                  _full((HIDDEN, HIDDEN)), _full((1, HIDDEN)),
                  _full((HIDDEN, PROJ)), _full((1, PROJ))],
        out_specs=[_row(PROJ), _row(HIDDEN)],
        out_shape=[jax.ShapeDtypeStruct((NT, PROJ), F32),
                   jax.ShapeDtypeStruct((NT, HIDDEN), F32)],
    )(h, p0, p1, dinv, uh, ua, bu.reshape(1, -1), pw0, pb0.reshape(1, -1),
      pw1, pb1.reshape(1, -1))


def kernel(x, positions, enc_W0, enc_b0, enc_W1, enc_b1, msg_W, msg_b,
           upd_W, upd_b, proj_W0, proj_b0, proj_W1, proj_b1):
    pos_t = positions.transpose(0, 2, 1)
    dst = _topk(pos_t)
    dst_flat = dst.reshape(NT * K)
    dst3 = dst_flat.reshape(NW, ANCHK, AECH)

    xf = x.reshape(NT, NODE_DIM)
    ws = [msg_W[i][:HIDDEN] for i in range(N_MP)]
    wr = [msg_W[i][HIDDEN:] for i in range(N_MP)]
    uh = [upd_W[i][:HIDDEN] for i in range(N_MP)]
    ua = [upd_W[i][HIDDEN:] for i in range(N_MP)]

    h, a, c = _encode(xf, enc_W0, enc_b0, enc_W1, enc_b1, ws[0], msg_b[0], wr[0])

    degp = _sc_deg(dst_flat)
    parts = _sc_agg(a, c, dst3)
    h, a, c, dinv = _update0(h, parts[0], parts[1], degp[0], degp[1],
                             uh[0], ua[0], upd_b[0], ws[1], msg_b[1], wr[1])

    parts = _sc_agg(a, c, dst3)
    h, a, c = _update1(h, parts[0], parts[1], dinv, uh[1], ua[1], upd_b[1],
                       ws[2], msg_b[2], wr[2])

    parts = _sc_agg(a, c, dst3)
    z, h = _update2(h, parts[0], parts[1], dinv, uh[2], ua[2], upd_b[2],
                    proj_W0, proj_b0, proj_W1, proj_b1)

    return z.reshape(B, N, PROJ), h.reshape(B, N, HIDDEN)
